# baseline (speedup 1.0000x reference)
import functools

import jax
import jax.numpy as jnp
from jax import lax
from jax.experimental import pallas as pl
from jax.experimental.pallas import tpu as pltpu
from jax.experimental.pallas import tpu_sc as plsc

N = 10000
C = 128
BM = 200
NBUF = 4
MT = 7400
MS = N - MT
NW = 32
NCHUNK = N // 16
BANK = 136
GU = 25
MAXR = -(-MS // NW)


def _tc_kernel(y_ref, mask_ref, prop_hbm, out_hbm, label_ref, flag_ref):
    flag_ref[0] = 0

    def step(prop_blk, out_blk):
        @pl.when(flag_ref[0] == 0)
        def _build_label():
            classes = jax.lax.broadcasted_iota(jnp.int32, (N, C), 1)
            eq = classes == y_ref[:][:, None]
            maskf = jnp.where(mask_ref[:], 1.0, 0.0)[:, None]
            label_ref[:] = jnp.where(eq, maskf, 0.0)
            flag_ref[0] = 1

        acc = jax.lax.dot_general(
            prop_blk[:],
            label_ref[:],
            (((1,), (0,)), ((), ())),
            preferred_element_type=jnp.float32,
        )
        out_blk[:] = jnp.clip(acc, 0.0, 1.0)

    pipeline = pltpu.emit_pipeline(
        step,
        grid=(MT // BM,),
        in_specs=[
            pl.BlockSpec((BM, N), lambda i: (i, 0),
                         pipeline_mode=pl.Buffered(buffer_count=NBUF)),
        ],
        out_specs=[pl.BlockSpec((BM, C), lambda i: (i, 0))],
    )
    pipeline(prop_hbm, out_hbm)


def _tc_part(y, train_mask, prop):
    return pl.pallas_call(
        _tc_kernel,
        in_specs=[
            pl.BlockSpec((N,), lambda: (0,)),
            pl.BlockSpec((N,), lambda: (0,)),
            pl.BlockSpec(memory_space=pl.ANY),
        ],
        out_specs=pl.BlockSpec(memory_space=pl.ANY),
        out_shape=jax.ShapeDtypeStruct((MT, C), jnp.float32),
        scratch_shapes=[pltpu.VMEM((N, C), jnp.float32),
                        pltpu.SMEM((1,), jnp.int32)],
    )(y, train_mask, prop)


def _sc_kernel(prop_hbm, y_hbm, maskf_hbm, out_hbm,
               y_v, mf_v, idx_v, buf0, buf1, acc, acc2, outrow, sem0, sem1):
    cid = lax.axis_index("c")
    sid = lax.axis_index("s")
    wid = sid * 2 + cid

    pltpu.sync_copy(y_hbm, y_v)
    pltpu.sync_copy(maskf_hbm, mf_v)
    lane_off = lax.iota(jnp.int32, 16) * BANK

    def mk_idx(j, _):
        yc = y_v[pl.ds(j * 16, 16)]
        mf = mf_v[pl.ds(j * 16, 16)]
        cls = jnp.where(mf > 0.5, yc, C)
        idx_v[pl.ds(j * 16, 16)] = cls + lane_off
        return 0

    lax.fori_loop(0, NCHUNK, mk_idx, 0)

    nw = (MS - wid + NW - 1) // NW
    row0 = MT + wid

    def compute_row(buf, t):
        zero = jnp.zeros((16,), jnp.float32)

        def z(j, _):
            acc[pl.ds(j * 16, 16)] = zero
            acc2[pl.ds(j * 16, 16)] = zero
            return 0

        lax.fori_loop(0, (16 * BANK) // 16, z, 0)

        def grp(g, _):
            base = g * (16 * GU)
            for u in range(GU):
                o = base + u * 16
                vals = buf[pl.ds(o, 16)]
                idx = idx_v[pl.ds(o, 16)]
                plsc.addupdate_scatter(acc if u % 2 == 0 else acc2, [idx], vals)
            return 0

        lax.fori_loop(0, NCHUNK // GU, grp, 0)

        for sb in range(C // 16):
            tot = acc[pl.ds(sb * 16, 16)] + acc2[pl.ds(sb * 16, 16)]
            for l in range(1, 16):
                tot = tot + acc[pl.ds(l * BANK + sb * 16, 16)]
                tot = tot + acc2[pl.ds(l * BANK + sb * 16, 16)]
            outrow[pl.ds(sb * 16, 16)] = tot
        pltpu.sync_copy(outrow, out_hbm.at[wid + NW * t])

    @pl.when(nw > 0)
    def _prime():
        pltpu.make_async_copy(prop_hbm.at[row0], buf0, sem0).start()

    def pair(u, _):
        t0 = 2 * u
        t1 = 2 * u + 1

        @pl.when(t0 < nw)
        def _even():
            pltpu.make_async_copy(prop_hbm.at[row0 + NW * t0], buf0, sem0).wait()

            @pl.when(t1 < nw)
            def _start_odd():
                pltpu.make_async_copy(
                    prop_hbm.at[row0 + NW * t1], buf1, sem1).start()

            compute_row(buf0, t0)

        @pl.when(t1 < nw)
        def _odd():
            pltpu.make_async_copy(prop_hbm.at[row0 + NW * t1], buf1, sem1).wait()

            @pl.when(t1 + 1 < nw)
            def _start_even():
                pltpu.make_async_copy(
                    prop_hbm.at[row0 + NW * (t1 + 1)], buf0, sem0).start()

            compute_row(buf1, t1)

        return 0

    lax.fori_loop(0, (MAXR + 1) // 2, pair, 0)


def _sc_part(y, maskf, prop):
    mesh = plsc.VectorSubcoreMesh(core_axis_name="c", subcore_axis_name="s")
    return pl.kernel(
        _sc_kernel,
        out_type=jax.ShapeDtypeStruct((MS, C), jnp.float32),
        mesh=mesh,
        scratch_types=[
            pltpu.VMEM((N,), jnp.int32),
            pltpu.VMEM((N,), jnp.float32),
            pltpu.VMEM((N,), jnp.int32),
            pltpu.VMEM((N,), jnp.float32),
            pltpu.VMEM((N,), jnp.float32),
            pltpu.VMEM((16 * BANK,), jnp.float32),
            pltpu.VMEM((16 * BANK,), jnp.float32),
            pltpu.VMEM((C,), jnp.float32),
            pltpu.SemaphoreType.DMA,
            pltpu.SemaphoreType.DMA,
        ],
        compiler_params=pltpu.CompilerParams(needs_layout_passes=False),
    )(prop, y, maskf)


@functools.partial(jax.jit, static_argnames=())
def kernel(x, y, train_mask, prop):
    del x
    maskf = train_mask.astype(jnp.float32)
    sc_out = _sc_part(y, maskf, prop)
    tc_out = _tc_part(y, train_mask, prop)
    sc_out = jnp.clip(sc_out, 0.0, 1.0)
    return jnp.concatenate([tc_out, sc_out], axis=0)


# speedup vs baseline: 3.0286x; 3.0286x over previous
import functools

import jax
import jax.numpy as jnp
from jax import lax
from jax.experimental import pallas as pl
from jax.experimental.pallas import tpu as pltpu
from jax.experimental.pallas import tpu_sc as plsc

N = 10000
C = 128
BM = 200
NBUF = 4
MT = 9400
MS = N - MT
NW = 32
NCHUNK = N // 16
BANK = 136
GU = 25
MAXR = -(-MS // NW)


def _tc_kernel(y_ref, mask_ref, prop_hbm, out_hbm, label_ref, flag_ref):
    flag_ref[0] = 0

    def step(prop_blk, out_blk):
        @pl.when(flag_ref[0] == 0)
        def _build_label():
            classes = jax.lax.broadcasted_iota(jnp.int32, (N, C), 1)
            eq = classes == y_ref[:][:, None]
            maskf = jnp.where(mask_ref[:], 1.0, 0.0)[:, None]
            label_ref[:] = jnp.where(eq, maskf, 0.0)
            flag_ref[0] = 1

        acc = jax.lax.dot_general(
            prop_blk[:],
            label_ref[:],
            (((1,), (0,)), ((), ())),
            preferred_element_type=jnp.float32,
        )
        out_blk[:] = jnp.clip(acc, 0.0, 1.0)

    pipeline = pltpu.emit_pipeline(
        step,
        grid=(MT // BM,),
        in_specs=[
            pl.BlockSpec((BM, N), lambda i: (i, 0),
                         pipeline_mode=pl.Buffered(buffer_count=NBUF)),
        ],
        out_specs=[pl.BlockSpec((BM, C), lambda i: (i, 0))],
    )
    pipeline(prop_hbm, out_hbm)


def _tc_part(y, train_mask, prop):
    return pl.pallas_call(
        _tc_kernel,
        in_specs=[
            pl.BlockSpec((N,), lambda: (0,)),
            pl.BlockSpec((N,), lambda: (0,)),
            pl.BlockSpec(memory_space=pl.ANY),
        ],
        out_specs=pl.BlockSpec(memory_space=pl.ANY),
        out_shape=jax.ShapeDtypeStruct((MT, C), jnp.float32),
        scratch_shapes=[pltpu.VMEM((N, C), jnp.float32),
                        pltpu.SMEM((1,), jnp.int32)],
    )(y, train_mask, prop)


def _sc_kernel(prop_hbm, y_hbm, maskf_hbm, out_hbm,
               y_v, mf_v, idx_v, buf0, buf1, acc, acc2, outrow, sem0, sem1):
    cid = lax.axis_index("c")
    sid = lax.axis_index("s")
    wid = sid * 2 + cid

    pltpu.sync_copy(y_hbm, y_v)
    pltpu.sync_copy(maskf_hbm, mf_v)
    lane_off = lax.iota(jnp.int32, 16) * BANK

    def mk_idx(j, _):
        yc = y_v[pl.ds(j * 16, 16)]
        mf = mf_v[pl.ds(j * 16, 16)]
        cls = jnp.where(mf > 0.5, yc, C)
        idx_v[pl.ds(j * 16, 16)] = cls + lane_off
        return 0

    lax.fori_loop(0, NCHUNK, mk_idx, 0)

    nw = (MS - wid + NW - 1) // NW
    row0 = MT + wid

    def compute_row(buf, t):
        zero = jnp.zeros((16,), jnp.float32)

        def z(j, _):
            acc[pl.ds(j * 16, 16)] = zero
            acc2[pl.ds(j * 16, 16)] = zero
            return 0

        lax.fori_loop(0, (16 * BANK) // 16, z, 0)

        def grp(g, _):
            base = g * (16 * GU)
            for u in range(GU):
                o = base + u * 16
                vals = buf[pl.ds(o, 16)]
                idx = idx_v[pl.ds(o, 16)]
                plsc.addupdate_scatter(acc if u % 2 == 0 else acc2, [idx], vals)
            return 0

        lax.fori_loop(0, NCHUNK // GU, grp, 0)

        for sb in range(C // 16):
            tot = acc[pl.ds(sb * 16, 16)] + acc2[pl.ds(sb * 16, 16)]
            for l in range(1, 16):
                tot = tot + acc[pl.ds(l * BANK + sb * 16, 16)]
                tot = tot + acc2[pl.ds(l * BANK + sb * 16, 16)]
            outrow[pl.ds(sb * 16, 16)] = jnp.clip(tot, 0.0, 1.0)
        pltpu.sync_copy(outrow, out_hbm.at[wid + NW * t])

    @pl.when(nw > 0)
    def _prime():
        pltpu.make_async_copy(prop_hbm.at[row0], buf0, sem0).start()

    def pair(u, _):
        t0 = 2 * u
        t1 = 2 * u + 1

        @pl.when(t0 < nw)
        def _even():
            pltpu.make_async_copy(prop_hbm.at[row0 + NW * t0], buf0, sem0).wait()

            @pl.when(t1 < nw)
            def _start_odd():
                pltpu.make_async_copy(
                    prop_hbm.at[row0 + NW * t1], buf1, sem1).start()

            compute_row(buf0, t0)

        @pl.when(t1 < nw)
        def _odd():
            pltpu.make_async_copy(prop_hbm.at[row0 + NW * t1], buf1, sem1).wait()

            @pl.when(t1 + 1 < nw)
            def _start_even():
                pltpu.make_async_copy(
                    prop_hbm.at[row0 + NW * (t1 + 1)], buf0, sem0).start()

            compute_row(buf1, t1)

        return 0

    lax.fori_loop(0, (MAXR + 1) // 2, pair, 0)


def _sc_part(y, maskf, prop):
    mesh = plsc.VectorSubcoreMesh(core_axis_name="c", subcore_axis_name="s")
    return pl.kernel(
        _sc_kernel,
        out_type=jax.ShapeDtypeStruct((MS, C), jnp.float32),
        mesh=mesh,
        scratch_types=[
            pltpu.VMEM((N,), jnp.int32),
            pltpu.VMEM((N,), jnp.float32),
            pltpu.VMEM((N,), jnp.int32),
            pltpu.VMEM((N,), jnp.float32),
            pltpu.VMEM((N,), jnp.float32),
            pltpu.VMEM((16 * BANK,), jnp.float32),
            pltpu.VMEM((16 * BANK,), jnp.float32),
            pltpu.VMEM((C,), jnp.float32),
            pltpu.SemaphoreType.DMA,
            pltpu.SemaphoreType.DMA,
        ],
        compiler_params=pltpu.CompilerParams(needs_layout_passes=False),
    )(prop, y, maskf)


@functools.partial(jax.jit, static_argnames=())
def kernel(x, y, train_mask, prop):
    del x
    maskf = train_mask.astype(jnp.float32)
    sc_out = _sc_part(y, maskf, prop)
    tc_out = _tc_part(y, train_mask, prop)
    return jnp.concatenate([tc_out, sc_out], axis=0)


# speedup vs baseline: 3.3866x; 1.1182x over previous
import functools

import jax
import jax.numpy as jnp
from jax.experimental import pallas as pl
from jax.experimental.pallas import tpu as pltpu

N = 10000
C = 128
BM = 200
NBUF = 6


def _lp_kernel(y_ref, mask_ref, prop_hbm, out_hbm, label_ref, flag_ref):
    flag_ref[0] = 0

    def step(prop_blk, out_blk):
        @pl.when(flag_ref[0] == 0)
        def _build_label():
            classes = jax.lax.broadcasted_iota(jnp.int32, (N, C), 1)
            eq = classes == y_ref[:][:, None]
            maskf = jnp.where(mask_ref[:], 1.0, 0.0)[:, None]
            label_ref[:] = jnp.where(eq, maskf, 0.0)
            flag_ref[0] = 1

        acc = jax.lax.dot_general(
            prop_blk[:],
            label_ref[:],
            (((1,), (0,)), ((), ())),
            preferred_element_type=jnp.float32,
        )
        out_blk[:] = jnp.clip(acc, 0.0, 1.0)

    pipeline = pltpu.emit_pipeline(
        step,
        grid=(N // BM,),
        in_specs=[
            pl.BlockSpec((BM, N), lambda i: (i, 0),
                         pipeline_mode=pl.Buffered(buffer_count=NBUF)),
        ],
        out_specs=[pl.BlockSpec((BM, C), lambda i: (i, 0))],
    )
    pipeline(prop_hbm, out_hbm)


@functools.partial(jax.jit, static_argnames=())
def kernel(x, y, train_mask, prop):
    del x
    return pl.pallas_call(
        _lp_kernel,
        in_specs=[
            pl.BlockSpec((N,), lambda: (0,)),
            pl.BlockSpec((N,), lambda: (0,)),
            pl.BlockSpec(memory_space=pl.ANY),
        ],
        out_specs=pl.BlockSpec(memory_space=pl.ANY),
        out_shape=jax.ShapeDtypeStruct((N, C), jnp.float32),
        scratch_shapes=[pltpu.VMEM((N, C), jnp.float32),
                        pltpu.SMEM((1,), jnp.int32)],
    )(y, train_mask, prop)


# speedup vs baseline: 3.4117x; 1.0074x over previous
import functools

import jax
import jax.numpy as jnp
from jax.experimental import pallas as pl
from jax.experimental.pallas import tpu as pltpu

N = 10000
C = 128
BM = 400
NBUF = 3


def _lp_kernel(y_ref, mask_ref, prop_hbm, out_hbm, label_ref, flag_ref):
    flag_ref[0] = 0

    def step(prop_blk, out_blk):
        @pl.when(flag_ref[0] == 0)
        def _build_label():
            classes = jax.lax.broadcasted_iota(jnp.int32, (N, C), 1)
            eq = classes == y_ref[:][:, None]
            maskf = jnp.where(mask_ref[:], 1.0, 0.0)[:, None]
            label_ref[:] = jnp.where(eq, maskf, 0.0)
            flag_ref[0] = 1

        acc = jax.lax.dot_general(
            prop_blk[:],
            label_ref[:],
            (((1,), (0,)), ((), ())),
            preferred_element_type=jnp.float32,
        )
        out_blk[:] = jnp.clip(acc, 0.0, 1.0)

    pipeline = pltpu.emit_pipeline(
        step,
        grid=(N // BM,),
        in_specs=[
            pl.BlockSpec((BM, N), lambda i: (i, 0),
                         pipeline_mode=pl.Buffered(buffer_count=NBUF)),
        ],
        out_specs=[pl.BlockSpec((BM, C), lambda i: (i, 0))],
    )
    pipeline(prop_hbm, out_hbm)


@functools.partial(jax.jit, static_argnames=())
def kernel(x, y, train_mask, prop):
    del x
    return pl.pallas_call(
        _lp_kernel,
        in_specs=[
            pl.BlockSpec((N,), lambda: (0,)),
            pl.BlockSpec((N,), lambda: (0,)),
            pl.BlockSpec(memory_space=pl.ANY),
        ],
        out_specs=pl.BlockSpec(memory_space=pl.ANY),
        out_shape=jax.ShapeDtypeStruct((N, C), jnp.float32),
        scratch_shapes=[pltpu.VMEM((N, C), jnp.float32),
                        pltpu.SMEM((1,), jnp.int32)],
    )(y, train_mask, prop)


# speedup vs baseline: 3.4654x; 1.0157x over previous
import functools

import jax
import jax.numpy as jnp
from jax.experimental import pallas as pl
from jax.experimental.pallas import tpu as pltpu

N = 10000
C = 128
BM = 200
NBUF = 4


def _lp_kernel(y_ref, mask_ref, prop_hbm, out_hbm, label_ref, flag_ref):
    flag_ref[0] = 0

    def step(prop_blk, out_blk):
        @pl.when(flag_ref[0] == 0)
        def _build_label():
            classes = jax.lax.broadcasted_iota(jnp.int32, (N, C), 1)
            eq = classes == y_ref[:][:, None]
            maskf = jnp.where(mask_ref[:], 1.0, 0.0)[:, None]
            label_ref[:] = jnp.where(eq, maskf, 0.0)
            flag_ref[0] = 1

        acc = jax.lax.dot_general(
            prop_blk[:],
            label_ref[:],
            (((1,), (0,)), ((), ())),
            preferred_element_type=jnp.float32,
        )
        out_blk[:] = jnp.clip(acc, 0.0, 1.0)

    pipeline = pltpu.emit_pipeline(
        step,
        grid=(N // BM,),
        in_specs=[
            pl.BlockSpec((BM, N), lambda i: (i, 0),
                         pipeline_mode=pl.Buffered(buffer_count=NBUF)),
        ],
        out_specs=[pl.BlockSpec((BM, C), lambda i: (i, 0))],
    )
    pipeline(prop_hbm, out_hbm)


@functools.partial(jax.jit, static_argnames=())
def kernel(x, y, train_mask, prop):
    del x
    return pl.pallas_call(
        _lp_kernel,
        in_specs=[
            pl.BlockSpec((N,), lambda: (0,)),
            pl.BlockSpec((N,), lambda: (0,)),
            pl.BlockSpec(memory_space=pl.ANY),
        ],
        out_specs=pl.BlockSpec(memory_space=pl.ANY),
        out_shape=jax.ShapeDtypeStruct((N, C), jnp.float32),
        scratch_shapes=[pltpu.VMEM((N, C), jnp.float32),
                        pltpu.SMEM((1,), jnp.int32)],
    )(y, train_mask, prop)
